# baseline (device time: 125830 ns/iter reference)
import jax
import jax.numpy as jnp
from jax import lax
from jax.experimental import pallas as pl
from jax.experimental.pallas import tpu as pltpu

N_DEV = 8
SQ = 1024
DH = 128
HQ_PER = 8
D_MODEL = 1024
WINDOW = 128
KBAND = 3 * 128
SCALE = 0.08838834764831843
CHUNK = SQ // N_DEV

PART_ROWS = (384, 384, 256)
PART_OFF = (0, 384, 768)
PART_MASKS = ((4, 3, 1), (3, 1, 4), (1, 4, 3))
N_ROUNDS = 3
MAX_HALF = 192


def kernel(x, Wq, K_ext, V_ext, Wo):
    x2 = x[0]
    k2 = K_ext[0].reshape(SQ, 64 * DH)
    v2 = V_ext[0].reshape(SQ, 64 * DH)

    def body(x_ref, wq_ref, k_any, v_any, wo_ref, out_ref,
             acc_ref, k_ref, v_ref, rs_buf, tx_buf, ag_buf,
             kv_sems, send_sems, recv_sems):
        my = lax.axis_index("i")

        def side(mask):
            if mask == 4:
                return (my // 4) % 2
            if mask == 3:
                return (my // 2) % 2
            return (my + my // 2) % 2

        kv_dmas = []
        for src, dst, sem in ((k_any, k_ref, kv_sems.at[0]),
                              (v_any, v_ref, kv_sems.at[1])):
            dma = pltpu.make_async_copy(
                src.at[:, pl.ds(my * D_MODEL, D_MODEL)], dst, sem)
            dma.start()
            kv_dmas.append(dma)

        barrier = pltpu.get_barrier_semaphore()
        for mask in (1, 3, 4):
            pl.semaphore_signal(barrier, inc=1, device_id=(my ^ mask,),
                                device_id_type=pl.DeviceIdType.MESH)
        pl.semaphore_wait(barrier, 3)

        q = jnp.dot(x_ref[...].astype(jnp.bfloat16),
                    wq_ref[...].astype(jnp.bfloat16),
                    preferred_element_type=jnp.float32) * SCALE
        qb = q.astype(jnp.bfloat16)
        wo_b = wo_ref[...].astype(jnp.bfloat16)

        for dma in kv_dmas:
            dma.wait()
        k_all = k_ref[...].astype(jnp.bfloat16)
        v_all = v_ref[...].astype(jnp.bfloat16)

        masks = []
        for rel in (0, 128, 256):
            qi = rel + lax.broadcasted_iota(jnp.int32, (CHUNK, KBAND), 0)
            ki = lax.broadcasted_iota(jnp.int32, (CHUNK, KBAND), 1)
            masks.append(jnp.abs(qi - ki) <= WINDOW)

        offs = [jnp.int32(0)] * 3
        sizes = list(PART_ROWS)
        pending = []
        part_last_chunk = {2: 0, 5: 1, 7: 2}
        started_r0 = [None] * 3

        def issue_rs(p, r):
            mask = PART_MASKS[p][r]
            half = sizes[p] // 2
            b = side(mask)
            send_off = PART_OFF[p] + offs[p] + (1 - b) * half
            keep_off = PART_OFF[p] + offs[p] + b * half
            tx_buf[p, r, :half, :] = (
                acc_ref[pl.ds(send_off, half), :].astype(jnp.bfloat16))
            rdma = pltpu.make_async_remote_copy(
                src_ref=tx_buf.at[p, r, pl.ds(0, half), :],
                dst_ref=rs_buf.at[p, r, pl.ds(0, half), :],
                send_sem=send_sems.at[p * 6 + r],
                recv_sem=recv_sems.at[p * 6 + r],
                device_id=(my ^ mask,),
                device_id_type=pl.DeviceIdType.MESH,
            )
            rdma.start()
            offs[p] = offs[p] + b * half
            sizes[p] = half
            return rdma, keep_off, half

        for cc in range(N_DEV):
            row0 = cc * CHUNK
            start = min(max(row0 - WINDOW, 0), SQ - KBAND)
            band = masks[(row0 - start) // CHUNK]
            ctx_parts = []
            for h in range(HQ_PER):
                s = lax.dot_general(qb[row0:row0 + CHUNK, h * DH:(h + 1) * DH],
                                    k_all[start:start + KBAND,
                                          h * DH:(h + 1) * DH],
                                    (((1,), (1,)), ((), ())),
                                    preferred_element_type=jnp.float32)
                w = jnp.exp(jnp.where(band, s, -1e9))
                ctx = jnp.dot(w.astype(jnp.bfloat16),
                              v_all[start:start + KBAND,
                                    h * DH:(h + 1) * DH],
                              preferred_element_type=jnp.float32)
                ctx = ctx / jnp.sum(w, axis=1, keepdims=True)
                ctx_parts.append(ctx.astype(jnp.bfloat16))
            ctx_c = jnp.concatenate(ctx_parts, axis=1)
            acc_ref[pl.ds(row0, CHUNK), :] = jnp.dot(
                ctx_c, wo_b, preferred_element_type=jnp.float32)
            if cc in part_last_chunk:
                p = part_last_chunk[cc]
                started_r0[p] = issue_rs(p, 0)

        for p, (rdma, keep_off, half) in enumerate(started_r0):
            rdma.wait_recv()
            sl = pl.ds(keep_off, half)
            acc_ref[sl, :] = (acc_ref[sl, :]
                              + rs_buf[p, 0, :half, :].astype(jnp.float32))
            pending.append(rdma)
        for r in range(1, N_ROUNDS):
            started = [issue_rs(p, r) for p in range(3)]
            for p, (rdma, keep_off, half) in enumerate(started):
                rdma.wait_recv()
                sl = pl.ds(keep_off, half)
                acc_ref[sl, :] = (acc_ref[sl, :]
                                  + rs_buf[p, r, :half, :].astype(jnp.float32))
                pending.append(rdma)

        for p in range(3):
            sl = pl.ds(PART_OFF[p] + offs[p], sizes[p])
            ag_buf[sl, :] = acc_ref[sl, :].astype(jnp.bfloat16)

        for j in range(N_ROUNDS):
            started = []
            for p in range(3):
                mask = PART_MASKS[p][N_ROUNDS - 1 - j]
                b = side(mask)
                cur = sizes[p]
                sl = pl.ds(PART_OFF[p] + offs[p], cur)
                rdma = pltpu.make_async_remote_copy(
                    src_ref=ag_buf.at[sl, :],
                    dst_ref=ag_buf.at[sl, :],
                    send_sem=send_sems.at[p * 6 + N_ROUNDS + j],
                    recv_sem=recv_sems.at[p * 6 + N_ROUNDS + j],
                    device_id=(my ^ mask,),
                    device_id_type=pl.DeviceIdType.MESH,
                )
                rdma.start()
                started.append(rdma)
                offs[p] = offs[p] - b * cur
                sizes[p] = 2 * cur
            for rdma in started:
                rdma.wait_recv()
                pending.append(rdma)

        out_ref[0, :, :] = ag_buf[...].astype(jnp.float32)

        for rdma in pending:
            rdma.wait_send()

    return pl.pallas_call(
        body,
        out_shape=jax.ShapeDtypeStruct((1, SQ, D_MODEL), jnp.float32),
        in_specs=[
            pl.BlockSpec(memory_space=pltpu.VMEM),
            pl.BlockSpec(memory_space=pltpu.VMEM),
            pl.BlockSpec(memory_space=pltpu.MemorySpace.HBM),
            pl.BlockSpec(memory_space=pltpu.MemorySpace.HBM),
            pl.BlockSpec(memory_space=pltpu.VMEM),
        ],
        out_specs=pl.BlockSpec(memory_space=pltpu.VMEM),
        scratch_shapes=[
            pltpu.VMEM((SQ, D_MODEL), jnp.float32),
            pltpu.VMEM((SQ, D_MODEL), jnp.float32),
            pltpu.VMEM((SQ, D_MODEL), jnp.float32),
            pltpu.VMEM((3, N_ROUNDS, MAX_HALF, D_MODEL), jnp.bfloat16),
            pltpu.VMEM((3, N_ROUNDS, MAX_HALF, D_MODEL), jnp.bfloat16),
            pltpu.VMEM((SQ, D_MODEL), jnp.bfloat16),
            pltpu.SemaphoreType.DMA((2,)),
            pltpu.SemaphoreType.DMA((18,)),
            pltpu.SemaphoreType.DMA((18,)),
        ],
        compiler_params=pltpu.CompilerParams(collective_id=0),
    )(x2, Wq, k2, v2, Wo)


# device time: 66107 ns/iter; 1.9034x vs baseline; 1.9034x over previous
import jax
import jax.numpy as jnp
from jax import lax
from jax.experimental import pallas as pl
from jax.experimental.pallas import tpu as pltpu

N_DEV = 8
SQ = 1024
DH = 128
HQ_PER = 8
D_MODEL = 1024
WINDOW = 128
KBAND = 3 * 128
SCALE = 0.08838834764831843
CHUNK = SQ // N_DEV

PART_ROWS = (384, 384, 256)
PART_OFF = (0, 384, 768)
PART_MASKS = ((4, 3, 1), (3, 1, 4), (1, 4, 3))
N_ROUNDS = 3
MAX_HALF = 192


def kernel(x, Wq, K_ext, V_ext, Wo):
    x2 = x[0]

    def body(x_ref, wq_ref, k_any, v_any, wo_ref, out_ref,
             acc_ref, k_ref, v_ref, rs_buf, tx_buf, ag_buf,
             kv_sems, send_sems, recv_sems):
        my = lax.axis_index("i")

        def side(mask):
            if mask == 4:
                return (my // 4) % 2
            if mask == 3:
                return (my // 2) % 2
            return (my + my // 2) % 2

        kv_dmas = []
        for h in range(HQ_PER):
            hh = my * HQ_PER + h
            for src, dst, sem in ((k_any, k_ref, kv_sems.at[0, h]),
                                  (v_any, v_ref, kv_sems.at[1, h])):
                dma = pltpu.make_async_copy(src.at[0, :, hh, :],
                                            dst.at[h], sem)
                dma.start()
                kv_dmas.append(dma)

        barrier = pltpu.get_barrier_semaphore()
        for mask in (1, 3, 4):
            pl.semaphore_signal(barrier, inc=1, device_id=(my ^ mask,),
                                device_id_type=pl.DeviceIdType.MESH)
        pl.semaphore_wait(barrier, 3)

        q = jnp.dot(x_ref[...].astype(jnp.bfloat16),
                    wq_ref[...].astype(jnp.bfloat16),
                    preferred_element_type=jnp.float32) * SCALE
        qb = q.astype(jnp.bfloat16)
        wo_b = wo_ref[...].astype(jnp.bfloat16)

        for dma in kv_dmas:
            dma.wait()
        k_all = k_ref[...].astype(jnp.bfloat16)
        v_all = v_ref[...].astype(jnp.bfloat16)

        masks = []
        for rel in (0, 128, 256):
            qi = rel + lax.broadcasted_iota(jnp.int32, (CHUNK, KBAND), 0)
            ki = lax.broadcasted_iota(jnp.int32, (CHUNK, KBAND), 1)
            masks.append(jnp.abs(qi - ki) <= WINDOW)

        offs = [jnp.int32(0)] * 3
        sizes = list(PART_ROWS)
        pending = []
        rs_rd = [[None] * 3 for _ in range(3)]
        ag_rd = [[None] * 3 for _ in range(3)]

        def rs_issue(p, r):
            mask = PART_MASKS[p][r]
            half = sizes[p] // 2
            b = side(mask)
            send_off = PART_OFF[p] + offs[p] + (1 - b) * half
            keep_off = PART_OFF[p] + offs[p] + b * half
            tx_buf[p, r, :half, :] = (
                acc_ref[pl.ds(send_off, half), :].astype(jnp.bfloat16))
            rdma = pltpu.make_async_remote_copy(
                src_ref=tx_buf.at[p, r, pl.ds(0, half), :],
                dst_ref=rs_buf.at[p, r, pl.ds(0, half), :],
                send_sem=send_sems.at[p * 6 + r],
                recv_sem=recv_sems.at[p * 6 + r],
                device_id=(my ^ mask,),
                device_id_type=pl.DeviceIdType.MESH,
            )
            rdma.start()
            rs_rd[p][r] = (rdma, keep_off, half)
            offs[p] = offs[p] + b * half
            sizes[p] = half

        def rs_wait(p, r):
            rdma, keep_off, half = rs_rd[p][r]
            rdma.wait_recv()
            sl = pl.ds(keep_off, half)
            acc_ref[sl, :] = (acc_ref[sl, :]
                              + rs_buf[p, r, :half, :].astype(jnp.float32))
            pending.append(rdma)

        def rs_step(p, r):
            rs_wait(p, r - 1)
            rs_issue(p, r)

        def ag_issue(p, j):
            if j == 0:
                sl0 = pl.ds(PART_OFF[p] + offs[p], sizes[p])
                ag_buf[sl0, :] = acc_ref[sl0, :].astype(jnp.bfloat16)
            mask = PART_MASKS[p][N_ROUNDS - 1 - j]
            b = side(mask)
            cur = sizes[p]
            sl = pl.ds(PART_OFF[p] + offs[p], cur)
            rdma = pltpu.make_async_remote_copy(
                src_ref=ag_buf.at[sl, :],
                dst_ref=ag_buf.at[sl, :],
                send_sem=send_sems.at[p * 6 + N_ROUNDS + j],
                recv_sem=recv_sems.at[p * 6 + N_ROUNDS + j],
                device_id=(my ^ mask,),
                device_id_type=pl.DeviceIdType.MESH,
            )
            rdma.start()
            ag_rd[p][j] = rdma
            offs[p] = offs[p] - b * cur
            sizes[p] = 2 * cur

        def ag_wait(p, j):
            ag_rd[p][j].wait_recv()
            pending.append(ag_rd[p][j])

        def ag_step(p, j):
            ag_wait(p, j - 1)
            ag_issue(p, j)

        post_chunk = {
            2: [(rs_issue, 0, 0)],
            4: [(rs_step, 0, 1)],
            5: [(rs_issue, 1, 0), (rs_step, 0, 2)],
            6: [(rs_wait, 0, 2), (ag_issue, 0, 0)],
        }
        for cc in range(N_DEV):
            row0 = cc * CHUNK
            start = min(max(row0 - WINDOW, 0), SQ - KBAND)
            band = masks[(row0 - start) // CHUNK]
            ctx_parts = []
            for h in range(HQ_PER):
                s = lax.dot_general(qb[row0:row0 + CHUNK, h * DH:(h + 1) * DH],
                                    k_all[h, start:start + KBAND, :],
                                    (((1,), (1,)), ((), ())),
                                    preferred_element_type=jnp.float32)
                w = jnp.exp(jnp.where(band, s, -1e9))
                ctx = jnp.dot(w.astype(jnp.bfloat16),
                              v_all[h, start:start + KBAND, :],
                              preferred_element_type=jnp.float32)
                ctx = ctx / jnp.sum(w, axis=1, keepdims=True)
                ctx_parts.append(ctx.astype(jnp.bfloat16))
            ctx_c = jnp.concatenate(ctx_parts, axis=1)
            acc_ref[pl.ds(row0, CHUNK), :] = jnp.dot(
                ctx_c, wo_b, preferred_element_type=jnp.float32)
            for fn, *args in post_chunk.get(cc, ()):
                fn(*args)

        rs_issue(2, 0)
        rs_step(1, 1)
        ag_step(0, 1)
        rs_step(2, 1)
        rs_step(1, 2)
        ag_step(0, 2)
        rs_step(2, 2)
        rs_wait(1, 2)
        ag_issue(1, 0)
        ag_wait(0, 2)
        rs_wait(2, 2)
        ag_issue(2, 0)
        ag_step(1, 1)
        ag_step(2, 1)
        ag_step(1, 2)
        ag_step(2, 2)
        ag_wait(1, 2)
        ag_wait(2, 2)

        out_ref[0, :, :] = ag_buf[...].astype(jnp.float32)

        for rdma in pending:
            rdma.wait_send()

    return pl.pallas_call(
        body,
        out_shape=jax.ShapeDtypeStruct((1, SQ, D_MODEL), jnp.float32),
        in_specs=[
            pl.BlockSpec(memory_space=pltpu.VMEM),
            pl.BlockSpec(memory_space=pltpu.VMEM),
            pl.BlockSpec(memory_space=pltpu.MemorySpace.HBM),
            pl.BlockSpec(memory_space=pltpu.MemorySpace.HBM),
            pl.BlockSpec(memory_space=pltpu.VMEM),
        ],
        out_specs=pl.BlockSpec(memory_space=pltpu.VMEM),
        scratch_shapes=[
            pltpu.VMEM((SQ, D_MODEL), jnp.float32),
            pltpu.VMEM((HQ_PER, SQ, DH), jnp.float32),
            pltpu.VMEM((HQ_PER, SQ, DH), jnp.float32),
            pltpu.VMEM((3, N_ROUNDS, MAX_HALF, D_MODEL), jnp.bfloat16),
            pltpu.VMEM((3, N_ROUNDS, MAX_HALF, D_MODEL), jnp.bfloat16),
            pltpu.VMEM((SQ, D_MODEL), jnp.bfloat16),
            pltpu.SemaphoreType.DMA((2, HQ_PER)),
            pltpu.SemaphoreType.DMA((18,)),
            pltpu.SemaphoreType.DMA((18,)),
        ],
        compiler_params=pltpu.CompilerParams(collective_id=0),
    )(x2, Wq, K_ext, V_ext, Wo)


# device time: 51254 ns/iter; 2.4550x vs baseline; 1.2898x over previous
import jax
import jax.numpy as jnp
from jax import lax
from jax.experimental import pallas as pl
from jax.experimental.pallas import tpu as pltpu

N_DEV = 8
SQ = 1024
DH = 128
HQ_PER = 8
D_MODEL = 1024
WINDOW = 128
KBAND = 3 * 128
SCALE = 0.08838834764831843
CHUNK = SQ // N_DEV

PART_ROWS = (384, 384, 256)
PART_OFF = (0, 384, 768)
PART_MASKS = ((4, 3, 1), (3, 1, 4), (1, 4, 3))
N_ROUNDS = 3
MAX_HALF = 192


def kernel(x, Wq, K_ext, V_ext, Wo):
    x2 = x[0]

    def body(x_ref, wq_ref, k_any, v_any, wo_ref, out_ref,
             acc_ref, k_ref, v_ref, rs_buf, tx_buf, ag_buf,
             kv_sems, send_sems, recv_sems):
        my = lax.axis_index("i")

        def side(mask):
            if mask == 4:
                return (my // 4) % 2
            if mask == 3:
                return (my // 2) % 2
            return (my + my // 2) % 2

        kv_dmas = []
        for h in range(HQ_PER):
            hh = my * HQ_PER + h
            for src, dst, sem in ((k_any, k_ref, kv_sems.at[0, h]),
                                  (v_any, v_ref, kv_sems.at[1, h])):
                dma = pltpu.make_async_copy(src.at[0, :, hh, :],
                                            dst.at[h], sem)
                dma.start()
                kv_dmas.append(dma)

        barrier = pltpu.get_barrier_semaphore()
        for mask in (1, 3, 4):
            pl.semaphore_signal(barrier, inc=1, device_id=(my ^ mask,),
                                device_id_type=pl.DeviceIdType.MESH)
        pl.semaphore_wait(barrier, 3)

        q = jnp.dot(x_ref[...].astype(jnp.bfloat16),
                    wq_ref[...].astype(jnp.bfloat16),
                    preferred_element_type=jnp.float32) * SCALE
        qb = q.astype(jnp.bfloat16)
        wo_b = wo_ref[...].astype(jnp.bfloat16)

        for dma in kv_dmas:
            dma.wait()
        k_all = k_ref[...].astype(jnp.bfloat16)
        v_all = v_ref[...].astype(jnp.bfloat16)

        BCHUNK, BBAND = 2 * CHUNK, 4 * CHUNK
        masks = []
        for rel in (0, 128, 256):
            qi = rel + lax.broadcasted_iota(jnp.int32, (BCHUNK, BBAND), 0)
            ki = lax.broadcasted_iota(jnp.int32, (BCHUNK, BBAND), 1)
            masks.append(jnp.abs(qi - ki) <= WINDOW)

        offs = [jnp.int32(0)] * 3
        sizes = list(PART_ROWS)
        pending = []
        part_last_chunk = {1: 0, 2: 1, 3: 2}
        started_r0 = [None] * 3

        def issue_rs(p, r):
            mask = PART_MASKS[p][r]
            half = sizes[p] // 2
            b = side(mask)
            send_off = PART_OFF[p] + offs[p] + (1 - b) * half
            keep_off = PART_OFF[p] + offs[p] + b * half
            tx_buf[p, r, :half, :] = (
                acc_ref[pl.ds(send_off, half), :].astype(jnp.bfloat16))
            rdma = pltpu.make_async_remote_copy(
                src_ref=tx_buf.at[p, r, pl.ds(0, half), :],
                dst_ref=rs_buf.at[p, r, pl.ds(0, half), :],
                send_sem=send_sems.at[p * 6 + r],
                recv_sem=recv_sems.at[p * 6 + r],
                device_id=(my ^ mask,),
                device_id_type=pl.DeviceIdType.MESH,
            )
            rdma.start()
            offs[p] = offs[p] + b * half
            sizes[p] = half
            return rdma, keep_off, half

        for cc in range(SQ // BCHUNK):
            row0 = cc * BCHUNK
            start = min(max(row0 - WINDOW, 0), SQ - BBAND)
            band = masks[(row0 - start) // CHUNK]
            ctx_parts = []
            for h in range(HQ_PER):
                s = lax.dot_general(qb[row0:row0 + BCHUNK, h * DH:(h + 1) * DH],
                                    k_all[h, start:start + BBAND, :],
                                    (((1,), (1,)), ((), ())),
                                    preferred_element_type=jnp.float32)
                w = jnp.exp(jnp.where(band, s, -1e9))
                ctx = jnp.dot(w.astype(jnp.bfloat16),
                              v_all[h, start:start + BBAND, :],
                              preferred_element_type=jnp.float32)
                ctx = ctx / jnp.sum(w, axis=1, keepdims=True)
                ctx_parts.append(ctx.astype(jnp.bfloat16))
            ctx_c = jnp.concatenate(ctx_parts, axis=1)
            acc_ref[pl.ds(row0, BCHUNK), :] = jnp.dot(
                ctx_c, wo_b, preferred_element_type=jnp.float32)
            if cc in part_last_chunk:
                p = part_last_chunk[cc]
                started_r0[p] = issue_rs(p, 0)

        for p, (rdma, keep_off, half) in enumerate(started_r0):
            rdma.wait_recv()
            sl = pl.ds(keep_off, half)
            acc_ref[sl, :] = (acc_ref[sl, :]
                              + rs_buf[p, 0, :half, :].astype(jnp.float32))
            pending.append(rdma)
        for r in range(1, N_ROUNDS):
            started = [issue_rs(p, r) for p in range(3)]
            for p, (rdma, keep_off, half) in enumerate(started):
                rdma.wait_recv()
                sl = pl.ds(keep_off, half)
                acc_ref[sl, :] = (acc_ref[sl, :]
                                  + rs_buf[p, r, :half, :].astype(jnp.float32))
                pending.append(rdma)

        for p in range(3):
            sl = pl.ds(PART_OFF[p] + offs[p], sizes[p])
            ag_buf[sl, :] = acc_ref[sl, :].astype(jnp.bfloat16)

        for j in range(N_ROUNDS):
            started = []
            for p in range(3):
                mask = PART_MASKS[p][N_ROUNDS - 1 - j]
                b = side(mask)
                cur = sizes[p]
                sl = pl.ds(PART_OFF[p] + offs[p], cur)
                rdma = pltpu.make_async_remote_copy(
                    src_ref=ag_buf.at[sl, :],
                    dst_ref=ag_buf.at[sl, :],
                    send_sem=send_sems.at[p * 6 + N_ROUNDS + j],
                    recv_sem=recv_sems.at[p * 6 + N_ROUNDS + j],
                    device_id=(my ^ mask,),
                    device_id_type=pl.DeviceIdType.MESH,
                )
                rdma.start()
                started.append(rdma)
                offs[p] = offs[p] - b * cur
                sizes[p] = 2 * cur
            for rdma in started:
                rdma.wait_recv()
                pending.append(rdma)

        out_ref[0, :, :] = ag_buf[...].astype(jnp.float32)

        for rdma in pending:
            rdma.wait_send()

    return pl.pallas_call(
        body,
        out_shape=jax.ShapeDtypeStruct((1, SQ, D_MODEL), jnp.float32),
        in_specs=[
            pl.BlockSpec(memory_space=pltpu.VMEM),
            pl.BlockSpec(memory_space=pltpu.VMEM),
            pl.BlockSpec(memory_space=pltpu.MemorySpace.HBM),
            pl.BlockSpec(memory_space=pltpu.MemorySpace.HBM),
            pl.BlockSpec(memory_space=pltpu.VMEM),
        ],
        out_specs=pl.BlockSpec(memory_space=pltpu.VMEM),
        scratch_shapes=[
            pltpu.VMEM((SQ, D_MODEL), jnp.float32),
            pltpu.VMEM((HQ_PER, SQ, DH), jnp.float32),
            pltpu.VMEM((HQ_PER, SQ, DH), jnp.float32),
            pltpu.VMEM((3, N_ROUNDS, MAX_HALF, D_MODEL), jnp.bfloat16),
            pltpu.VMEM((3, N_ROUNDS, MAX_HALF, D_MODEL), jnp.bfloat16),
            pltpu.VMEM((SQ, D_MODEL), jnp.bfloat16),
            pltpu.SemaphoreType.DMA((2, HQ_PER)),
            pltpu.SemaphoreType.DMA((18,)),
            pltpu.SemaphoreType.DMA((18,)),
        ],
        compiler_params=pltpu.CompilerParams(collective_id=0),
    )(x2, Wq, K_ext, V_ext, Wo)


# device time: 50265 ns/iter; 2.5033x vs baseline; 1.0197x over previous
import jax
import jax.numpy as jnp
from jax import lax
from jax.experimental import pallas as pl
from jax.experimental.pallas import tpu as pltpu

N_DEV = 8
SQ = 1024
DH = 128
HQ_PER = 8
D_MODEL = 1024
WINDOW = 128
KBAND = 3 * 128
SCALE = 0.08838834764831843
CHUNK = SQ // N_DEV

PART_ROWS = (384, 384, 256)
PART_OFF = (0, 384, 768)
PART_MASKS = ((4, 3, 1), (3, 1, 4), (1, 4, 3))
N_ROUNDS = 3
MAX_HALF = 192


def kernel(x, Wq, K_ext, V_ext, Wo):
    x2 = x[0]

    def body(x_ref, wq_ref, k_any, v_any, wo_ref, out_ref,
             acc_ref, k_ref, v_ref, rs_buf, tx_buf, ag_buf,
             kv_sems, send_sems, recv_sems):
        my = lax.axis_index("i")

        def side(mask):
            if mask == 4:
                return (my // 4) % 2
            if mask == 3:
                return (my // 2) % 2
            return (my + my // 2) % 2

        kv_dmas = []
        for h in range(HQ_PER):
            hh = my * HQ_PER + h
            for src, dst, sem in ((k_any, k_ref, kv_sems.at[0, h]),
                                  (v_any, v_ref, kv_sems.at[1, h])):
                dma = pltpu.make_async_copy(src.at[0, :, hh, :],
                                            dst.at[h], sem)
                dma.start()
                kv_dmas.append(dma)

        barrier = pltpu.get_barrier_semaphore()
        for mask in (1, 3, 4):
            pl.semaphore_signal(barrier, inc=1, device_id=(my ^ mask,),
                                device_id_type=pl.DeviceIdType.MESH)
        pl.semaphore_wait(barrier, 3)

        q = jnp.dot(x_ref[...].astype(jnp.bfloat16),
                    wq_ref[...].astype(jnp.bfloat16),
                    preferred_element_type=jnp.float32) * SCALE
        qb = q.astype(jnp.bfloat16)
        wo_b = wo_ref[...].astype(jnp.bfloat16)

        for dma in kv_dmas:
            dma.wait()
        k_all = k_ref[...].astype(jnp.bfloat16)
        v_all = v_ref[...].astype(jnp.bfloat16)

        BCHUNK, BBAND = 2 * CHUNK, 4 * CHUNK
        masks = []
        for rel in (0, 128, 256):
            qi = rel + lax.broadcasted_iota(jnp.int32, (BCHUNK, BBAND), 0)
            ki = lax.broadcasted_iota(jnp.int32, (BCHUNK, BBAND), 1)
            masks.append(jnp.abs(qi - ki) <= WINDOW)

        offs = [jnp.int32(0)] * 3
        sizes = list(PART_ROWS)
        pending = []
        part_last_chunk = {1: 0, 2: 1, 3: 2}
        started_r0 = [None] * 3

        def issue_rs(p, r):
            mask = PART_MASKS[p][r]
            half = sizes[p] // 2
            b = side(mask)
            send_off = PART_OFF[p] + offs[p] + (1 - b) * half
            keep_off = PART_OFF[p] + offs[p] + b * half
            tx_buf[p, r, :half, :] = (
                acc_ref[pl.ds(send_off, half), :].astype(jnp.bfloat16))
            rdma = pltpu.make_async_remote_copy(
                src_ref=tx_buf.at[p, r, pl.ds(0, half), :],
                dst_ref=rs_buf.at[p, r, pl.ds(0, half), :],
                send_sem=send_sems.at[p * 6 + r],
                recv_sem=recv_sems.at[p * 6 + r],
                device_id=(my ^ mask,),
                device_id_type=pl.DeviceIdType.MESH,
            )
            rdma.start()
            offs[p] = offs[p] + b * half
            sizes[p] = half
            return rdma, keep_off, half

        for cc in range(SQ // BCHUNK):
            row0 = cc * BCHUNK
            start = min(max(row0 - WINDOW, 0), SQ - BBAND)
            band = masks[(row0 - start) // CHUNK]
            ctx_parts = []
            for h in range(HQ_PER):
                s = lax.dot_general(qb[row0:row0 + BCHUNK, h * DH:(h + 1) * DH],
                                    k_all[h, start:start + BBAND, :],
                                    (((1,), (1,)), ((), ())),
                                    preferred_element_type=jnp.float32)
                w = jnp.exp(jnp.where(band, s, -1e9))
                ctx = jnp.dot(w.astype(jnp.bfloat16),
                              v_all[h, start:start + BBAND, :],
                              preferred_element_type=jnp.float32)
                ctx = ctx / jnp.sum(w, axis=1, keepdims=True)
                ctx_parts.append(ctx.astype(jnp.bfloat16))
            ctx_c = jnp.concatenate(ctx_parts, axis=1)
            acc_ref[pl.ds(row0, BCHUNK), :] = jnp.dot(
                ctx_c, wo_b, preferred_element_type=jnp.float32)
            if cc in part_last_chunk:
                p = part_last_chunk[cc]
                started_r0[p] = issue_rs(p, 0)

        for p, (rdma, keep_off, half) in enumerate(started_r0):
            rdma.wait_recv()
            sl = pl.ds(keep_off, half)
            acc_ref[sl, :] = (acc_ref[sl, :]
                              + rs_buf[p, 0, :half, :].astype(jnp.float32))
            pending.append(rdma)
        for r in range(1, N_ROUNDS):
            started = []
            for p in range(3):
                mask = PART_MASKS[p][r]
                cur = sizes[p]
                sl = pl.ds(PART_OFF[p] + offs[p], cur)
                tx_buf[p, r, :cur, :] = acc_ref[sl, :].astype(jnp.bfloat16)
                rdma = pltpu.make_async_remote_copy(
                    src_ref=tx_buf.at[p, r, pl.ds(0, cur), :],
                    dst_ref=rs_buf.at[p, r, pl.ds(0, cur), :],
                    send_sem=send_sems.at[p * 6 + r],
                    recv_sem=recv_sems.at[p * 6 + r],
                    device_id=(my ^ mask,),
                    device_id_type=pl.DeviceIdType.MESH,
                )
                rdma.start()
                started.append((rdma, sl, cur))
            for p, (rdma, sl, cur) in enumerate(started):
                rdma.wait_recv()
                acc_ref[sl, :] = (acc_ref[sl, :]
                                  + rs_buf[p, r, :cur, :].astype(jnp.float32))
                pending.append(rdma)

        started = []
        for p in range(3):
            mask = PART_MASKS[p][0]
            cur = sizes[p]
            sl = pl.ds(PART_OFF[p] + offs[p], cur)
            ag_buf[sl, :] = acc_ref[sl, :].astype(jnp.bfloat16)
            rdma = pltpu.make_async_remote_copy(
                src_ref=ag_buf.at[sl, :],
                dst_ref=ag_buf.at[sl, :],
                send_sem=send_sems.at[p * 6 + 3],
                recv_sem=recv_sems.at[p * 6 + 3],
                device_id=(my ^ mask,),
                device_id_type=pl.DeviceIdType.MESH,
            )
            rdma.start()
            started.append(rdma)
        for rdma in started:
            rdma.wait_recv()
            pending.append(rdma)

        out_ref[0, :, :] = ag_buf[...].astype(jnp.float32)

        for rdma in pending:
            rdma.wait_send()

    return pl.pallas_call(
        body,
        out_shape=jax.ShapeDtypeStruct((1, SQ, D_MODEL), jnp.float32),
        in_specs=[
            pl.BlockSpec(memory_space=pltpu.VMEM),
            pl.BlockSpec(memory_space=pltpu.VMEM),
            pl.BlockSpec(memory_space=pltpu.MemorySpace.HBM),
            pl.BlockSpec(memory_space=pltpu.MemorySpace.HBM),
            pl.BlockSpec(memory_space=pltpu.VMEM),
        ],
        out_specs=pl.BlockSpec(memory_space=pltpu.VMEM),
        scratch_shapes=[
            pltpu.VMEM((SQ, D_MODEL), jnp.float32),
            pltpu.VMEM((HQ_PER, SQ, DH), jnp.float32),
            pltpu.VMEM((HQ_PER, SQ, DH), jnp.float32),
            pltpu.VMEM((3, N_ROUNDS, MAX_HALF, D_MODEL), jnp.bfloat16),
            pltpu.VMEM((3, N_ROUNDS, MAX_HALF, D_MODEL), jnp.bfloat16),
            pltpu.VMEM((SQ, D_MODEL), jnp.bfloat16),
            pltpu.SemaphoreType.DMA((2, HQ_PER)),
            pltpu.SemaphoreType.DMA((18,)),
            pltpu.SemaphoreType.DMA((18,)),
        ],
        compiler_params=pltpu.CompilerParams(collective_id=0),
    )(x2, Wq, K_ext, V_ext, Wo)
